# baseline (speedup 1.0000x reference)
import jax
import jax.numpy as jnp
from jax.experimental import pallas as pl
from jax.experimental.pallas import tpu as pltpu


def _scatter_body(idx_ref, val_ref, in_hbm_ref, out_ref):
    del idx_ref, in_hbm_ref
    out_ref[...] = val_ref[...]


def kernel(input, index, value):
    M, d = input.shape
    B = index.shape[0]
    idx = index.astype(jnp.int32)
    inp3 = input.reshape(M, 1, d)
    val3 = value.reshape(B, 1, d)

    grid_spec = pltpu.PrefetchScalarGridSpec(
        num_scalar_prefetch=1,
        grid=(B,),
        in_specs=[
            pl.BlockSpec((1, 1, d), lambda b, idx_ref: (b, 0, 0)),
            pl.BlockSpec(memory_space=pl.ANY),
        ],
        out_specs=pl.BlockSpec((1, 1, d), lambda b, idx_ref: (idx_ref[b], 0, 0)),
    )

    out = pl.pallas_call(
        _scatter_body,
        grid_spec=grid_spec,
        out_shape=jax.ShapeDtypeStruct((M, 1, d), input.dtype),
        input_output_aliases={2: 0},
    )(idx, val3, inp3)
    return out.reshape(M, d)


# speedup vs baseline: 4.1634x; 4.1634x over previous
import jax
import jax.numpy as jnp
from jax import lax
from jax.experimental import pallas as pl
from jax.experimental.pallas import tpu as pltpu
from jax.experimental.pallas import tpu_sc as plsc

_M, _D, _B = 100000, 128, 16384
_NC, _NS, _L = 2, 16, 16
_NW = _NC * _NS
_RPW = 3128
_RLAST = _M - (_NW - 1) * _RPW
_NCHUNK = _B // _L
_R = 64
_FCAP = _RPW + _R + _L

_CTZ_TAB = (0, 1, 28, 2, 29, 14, 24, 3, 30, 22, 20, 15, 25, 17, 4, 8,
            31, 27, 13, 23, 21, 19, 16, 7, 26, 12, 18, 6, 11, 5, 10, 9)


def _extract_at(v, sl):
    out = jnp.int32(0)
    for t in range(_L):
        out = jnp.where(sl == t, v[t], out)
    return out


def _body(in_hbm, idx_hbm, val_hbm, out_hbm,
          idx_v, wl, fold, tbl, fb_i, fb_b,
          iba, bba, ibb, bbb, rows_a, rows_b,
          sem_c, sem_i, sem_ga, sem_gb, sem_sa, sem_sb):
    wid = lax.axis_index("s") * _NC + lax.axis_index("c")
    lo = pl.multiple_of(wid * _RPW, 8)
    hi = jnp.minimum(lo + _RPW, _M)
    lane = lax.iota(jnp.int32, _L)
    zeros = jnp.zeros((_L,), jnp.int32)

    def _wait_copy():
        @pl.when(wid < _NW - 1)
        def _():
            pltpu.make_async_copy(in_hbm.at[pl.ds(lo, _RPW)],
                                  out_hbm.at[pl.ds(lo, _RPW)], sem_c).wait()

        @pl.when(wid == _NW - 1)
        def _():
            pltpu.make_async_copy(in_hbm.at[pl.ds(lo, _RLAST)],
                                  out_hbm.at[pl.ds(lo, _RLAST)], sem_c).wait()

    idx_dma = pltpu.make_async_copy(idx_hbm, idx_v, sem_i)
    idx_dma.start()

    @pl.when(wid < _NW - 1)
    def _copy_main():
        pltpu.make_async_copy(in_hbm.at[pl.ds(lo, _RPW)],
                              out_hbm.at[pl.ds(lo, _RPW)], sem_c).start()

    @pl.when(wid == _NW - 1)
    def _copy_last():
        pltpu.make_async_copy(in_hbm.at[pl.ds(lo, _RLAST)],
                              out_hbm.at[pl.ds(lo, _RLAST)], sem_c).start()

    neg1 = jnp.full((_L,), -1, jnp.int32)

    def wl_init(j, _):
        for u in range(8):
            wl[pl.ds((j * 8 + u) * _L, _L)] = neg1
        return 0

    lax.fori_loop(0, (_RPW + _L) // 8, wl_init, 0)
    for t, v in enumerate(_CTZ_TAB):
        tbl[pl.ds(t * _L, _L)] = jnp.full((_L,), v, jnp.int32)
    fold[pl.ds(_L, _L)] = zeros

    idx_dma.wait()

    two_pow = jnp.left_shift(jnp.int32(1), lane)

    def scan_step(c, _):
        iv = idx_v[pl.ds(c * _L, _L)]
        m = (iv >= lo) & (iv < hi)
        v = jnp.where(m, two_pow, 0)
        for k in (8, 4, 2, 1):
            fold[pl.ds(0, _L)] = v
            v = v | fold[pl.ds(k, _L)]
        bm = v[0]
        x = bm - ((bm >> 1) & 0x5555)
        x = (x & 0x3333) + ((x >> 2) & 0x3333)
        x = (x + (x >> 4)) & 0x0F0F
        cnt = (x + (x >> 8)) & 0x1F

        def visit(k2, bmc):
            low = bmc & (-bmc)
            h = lax.shift_right_logical(low * jnp.int32(0x077CB531), 27) & 31
            t = tbl[pl.ds(h * _L, _L)][0]
            e = idx_v[pl.ds(c * _L + t, _L)][0]
            wl[pl.ds((e - lo) * _L, _L)] = jnp.full(
                (_L,), c * _L + t, jnp.int32)
            return bmc & (bmc - 1)

        lax.fori_loop(0, cnt, visit, bm)
        return 0

    lax.fori_loop(0, _NCHUNK, scan_step, 0)

    nrows = hi - lo

    def sweep(r, carry):
        fk, ai, ab = carry
        wv = wl[pl.ds(r * _L, _L)]
        w = wv[0]
        keep = w >= 0
        sl = fk % _L
        ins = (wv >= 0) & (lane == sl)
        ai = jnp.where(ins, lo + r, ai)
        ab = jnp.where(ins, w, ab)

        @pl.when(keep & (sl == _L - 1))
        def _flush():
            fb_i[pl.ds((fk // _L) * _L, _L)] = ai
            fb_b[pl.ds((fk // _L) * _L, _L)] = ab

        return (fk + keep.astype(jnp.int32), ai, ab)

    fk, ai_f, ab_f = lax.fori_loop(0, nrows, sweep,
                                   (jnp.int32(0), zeros, zeros))

    @pl.when(fk > 0)
    def _apply():
        sl = (fk - 1) % _L
        li = _extract_at(ai_f, sl)
        lb = _extract_at(ab_f, sl)
        li_v = jnp.full((_L,), li, jnp.int32)
        lb_v = jnp.full((_L,), lb, jnp.int32)

        @pl.when(fk % _L > 0)
        def _flush_tail():
            fb_i[pl.ds((fk // _L) * _L, _L)] = jnp.where(
                lane <= sl, ai_f, li_v)
            fb_b[pl.ds((fk // _L) * _L, _L)] = jnp.where(
                lane <= sl, ab_f, lb_v)

        fk_r = ((fk + _L - 1) // _L) * _L
        fk_pad = ((fk + _R - 1) // _R) * _R

        def pad_body(j, _):
            fb_i[pl.ds(fk_r + j * _L, _L)] = li_v
            fb_b[pl.ds(fk_r + j * _L, _L)] = lb_v
            return 0

        lax.fori_loop(0, (fk_pad - fk_r) // _L, pad_body, 0)

        _wait_copy()

        nb = fk_pad // _R

        def _fire(g, ibuf, bbuf, rows, sem_g, sem_s):
            for t in range(_R // _L):
                ibuf[pl.ds(t * _L, _L)] = fb_i[pl.ds(g * _R + t * _L, _L)]
                bbuf[pl.ds(t * _L, _L)] = fb_b[pl.ds(g * _R + t * _L, _L)]
            gd = pltpu.make_async_copy(val_hbm.at[bbuf], rows, sem_g)
            gd.start()
            gd.wait()
            pltpu.make_async_copy(rows, out_hbm.at[ibuf], sem_s).start()

        def batch_body(g, _):
            even = g % 2 == 0

            @pl.when(even & (g >= 2))
            def _wa():
                pltpu.make_async_copy(rows_a, out_hbm.at[iba], sem_sa).wait()

            @pl.when(jnp.logical_not(even) & (g >= 2))
            def _wb():
                pltpu.make_async_copy(rows_b, out_hbm.at[ibb], sem_sb).wait()

            @pl.when(even)
            def _fa():
                _fire(g, iba, bba, rows_a, sem_ga, sem_sa)

            @pl.when(jnp.logical_not(even))
            def _fb():
                _fire(g, ibb, bbb, rows_b, sem_gb, sem_sb)

            return 0

        lax.fori_loop(0, nb, batch_body, 0)

        @pl.when(nb >= 2)
        def _drain_prev():
            even0 = nb % 2 == 0

            @pl.when(even0)
            def _():
                pltpu.make_async_copy(rows_a, out_hbm.at[iba], sem_sa).wait()

            @pl.when(jnp.logical_not(even0))
            def _():
                pltpu.make_async_copy(rows_b, out_hbm.at[ibb], sem_sb).wait()

        even1 = (nb - 1) % 2 == 0

        @pl.when(even1)
        def _drain_a():
            pltpu.make_async_copy(rows_a, out_hbm.at[iba], sem_sa).wait()

        @pl.when(jnp.logical_not(even1))
        def _drain_b():
            pltpu.make_async_copy(rows_b, out_hbm.at[ibb], sem_sb).wait()

    @pl.when(fk == 0)
    def _no_updates():
        _wait_copy()


def kernel(input, index, value):
    M, d = input.shape
    B = index.shape[0]
    assert (M, d, B) == (_M, _D, _B)
    idx = index.astype(jnp.int32)

    mesh = plsc.VectorSubcoreMesh(core_axis_name="c", subcore_axis_name="s")
    run = pl.kernel(
        _body,
        mesh=mesh,
        out_type=jax.ShapeDtypeStruct((M, d), jnp.float32),
        scratch_types=[
            pltpu.VMEM((_B,), jnp.int32),
            pltpu.VMEM(((_RPW + _L) * _L,), jnp.int32),
            pltpu.VMEM((2 * _L,), jnp.int32),
            pltpu.VMEM((32 * _L,), jnp.int32),
            pltpu.VMEM((_FCAP,), jnp.int32),
            pltpu.VMEM((_FCAP,), jnp.int32),
            pltpu.VMEM((_R,), jnp.int32),
            pltpu.VMEM((_R,), jnp.int32),
            pltpu.VMEM((_R,), jnp.int32),
            pltpu.VMEM((_R,), jnp.int32),
            pltpu.VMEM((_R, _D), jnp.float32),
            pltpu.VMEM((_R, _D), jnp.float32),
            pltpu.SemaphoreType.DMA,
            pltpu.SemaphoreType.DMA,
            pltpu.SemaphoreType.DMA,
            pltpu.SemaphoreType.DMA,
            pltpu.SemaphoreType.DMA,
            pltpu.SemaphoreType.DMA,
        ],
    )
    return run(input, idx, value)
